# baseline (speedup 1.0000x reference)
import functools

import numpy as np
import jax
import jax.numpy as jnp
from jax import lax
from jax.experimental import pallas as pl
from jax.experimental.pallas import tpu as pltpu
from jax.experimental.pallas import tpu_sc as plsc

EMB_DIM = 64
SPE1_IDX = 2
SPE2_IDX = 3
SEQ = 200
BATCH = 1024
TOK = BATCH * SEQ
NPROWS = 80
TOTAL_IDX = TOK + NPROWS

NC, NS = 2, 16
NW = NC * NS
CHUNK = 128
K = 51
BPW = CHUNK * K
BPAD = BPW * NW

BB = 16
ROWS_BLK = BB * SEQ
GRID = BATCH // BB


def _positional_encoding(L, d):
    position = np.arange(L, dtype=np.float32)[:, None]
    div_term = np.exp(np.arange(0, d, 2, dtype=np.float32) * (-np.log(10000.0) / d))
    pe = np.zeros((L, d), dtype=np.float32)
    pe[:, 0::2] = np.sin(position * div_term)
    pe[:, 1::2] = np.cos(position * div_term)
    return pe


_PE_REP = jnp.asarray(np.tile(_positional_encoding(SEQ, EMB_DIM), (BB, 1)))


def _sc_gather(emb_table, idx):
    mesh = plsc.VectorSubcoreMesh(core_axis_name="c", subcore_axis_name="s")

    @functools.partial(
        pl.kernel,
        mesh=mesh,
        out_type=jax.ShapeDtypeStruct((BPAD, EMB_DIM), jnp.float32),
        scratch_types=[
            pltpu.VMEM((K, CHUNK), jnp.int32),
            pltpu.VMEM((CHUNK, EMB_DIM), jnp.float32),
            pltpu.SemaphoreType.DMA,
        ],
        compiler_params=pltpu.CompilerParams(use_tc_tiling_on_sc=False),
    )
    def gather_kernel(table_hbm, idx_hbm, out_hbm, idx_v, rows_v, sem):
        wid = lax.axis_index("s") * NC + lax.axis_index("c")
        base = wid * BPW
        pltpu.sync_copy(idx_hbm.at[wid], idx_v)

        def body(j, carry):
            pltpu.async_copy(table_hbm.at[idx_v.at[j]], rows_v, sem).wait()
            pltpu.sync_copy(rows_v, out_hbm.at[pl.ds(base + j * CHUNK, CHUNK)])
            return carry

        lax.fori_loop(0, K, body, 0)

    return gather_kernel(emb_table, idx)


def _tc_body(emb_ref, seg_ref, prow_ref, pe_ref, w_ref, b_ref, out_ref):
    p0 = jnp.sum(prow_ref[0:40, :], axis=0, keepdims=True)
    p1 = jnp.sum(prow_ref[40:80, :], axis=0, keepdims=True)
    emb = emb_ref[...] * np.float32(8.0)
    seg = seg_ref[...]
    m0 = (seg == SPE1_IDX).astype(jnp.float32)
    m1 = (seg == SPE2_IDX).astype(jnp.float32)
    emb = emb + m0 * p0 + m1 * p1 + pe_ref[...]
    out_ref[...] = (
        jnp.dot(emb, w_ref[...], preferred_element_type=jnp.float32) + b_ref[...]
    )


def _tc_project(gathered, segs_col, proj_w, proj_b2):
    return pl.pallas_call(
        _tc_body,
        grid=(GRID,),
        in_specs=[
            pl.BlockSpec((ROWS_BLK, EMB_DIM), lambda i: (i, 0)),
            pl.BlockSpec((ROWS_BLK, 1), lambda i: (i, 0)),
            pl.BlockSpec((NPROWS, EMB_DIM), lambda i: (TOK // NPROWS, 0)),
            pl.BlockSpec((ROWS_BLK, EMB_DIM), lambda i: (0, 0)),
            pl.BlockSpec((EMB_DIM, 512), lambda i: (0, 0)),
            pl.BlockSpec((1, 512), lambda i: (0, 0)),
        ],
        out_specs=pl.BlockSpec((ROWS_BLK, 512), lambda i: (i, 0)),
        out_shape=jax.ShapeDtypeStruct((TOK, 512), jnp.float32),
    )(gathered, segs_col, gathered, _PE_REP, proj_w, proj_b2)


def kernel(context, segs, personas_no_tag, tags, emb_table, proj_w, proj_b):
    idx_full = jnp.concatenate([
        context.reshape(-1),
        personas_no_tag[0], tags[0],
        personas_no_tag[1], tags[1],
    ]).astype(jnp.int32)
    idx_pad = jnp.concatenate(
        [idx_full, jnp.zeros((BPAD - TOTAL_IDX,), jnp.int32)]
    ).reshape(NW, K, CHUNK)

    gathered = _sc_gather(emb_table, idx_pad)
    segs_col = segs.reshape(TOK, 1).astype(jnp.int32)
    out = _tc_project(gathered, segs_col, proj_w, proj_b.reshape(1, 512))
    return out.reshape(BATCH, SEQ, 512)
